# baseline (speedup 1.0000x reference)
import functools

import jax
import jax.numpy as jnp
from jax import lax
from jax.experimental import pallas as pl
from jax.experimental.pallas import tpu as pltpu
from jax.experimental.pallas import tpu_sc as plsc

NC, NS, L = 2, 16, 16
NW = NC * NS
B, T, D = 256, 300, 512
ROWS = B * T
RPW = ROWS // NW
TCH = 40
P2 = 2 * T
CPW = RPW // TCH
NBG = 2
NBP = 3
STEP = 6

_mesh = plsc.VectorSubcoreMesh(core_axis_name="c", subcore_axis_name="s")


@functools.partial(
    pl.kernel,
    out_type=jax.ShapeDtypeStruct((ROWS, D), jnp.float32),
    mesh=_mesh,
    scratch_types=[
        pltpu.VMEM((CPW, TCH), jnp.int32),
        pltpu.VMEM((NBG, TCH, D), jnp.float32),
        pltpu.VMEM((NBP, TCH, D), jnp.float32),
        pltpu.VMEM_SHARED((P2, D), jnp.float32),
        pltpu.SemaphoreType.DMA((NBG,)),
        pltpu.SemaphoreType.DMA((NBP,)),
        pltpu.SemaphoreType.DMA((NBP,)),
    ],
)
def _embed(idx_hbm, tok_hbm, pos2_hbm, out_hbm,
           idx_v, rows_in, pos_v, pos_sh, gsem, psem, ssem):
    sid = lax.axis_index("s")
    wid = sid * NC + lax.axis_index("c")
    base = wid * RPW

    @pl.when(sid == 0)
    def _():
        pltpu.sync_copy(pos2_hbm, pos_sh)
    plsc.subcore_barrier()

    pltpu.sync_copy(idx_hbm.at[wid], idx_v)

    def make_gather(k, bg):
        return pltpu.make_async_copy(
            tok_hbm.at[idx_v.at[k]], rows_in.at[bg], gsem.at[bg])

    def make_poscopy(k, bp):
        po = pl.multiple_of(lax.rem(k * TCH, P2), 8)
        return pltpu.make_async_copy(
            pos_sh.at[pl.ds(po, TCH), :], pos_v.at[bp], psem.at[bp])

    def make_store(k, bp):
        off = pl.multiple_of(base + k * TCH, 8)
        return pltpu.make_async_copy(
            pos_v.at[bp], out_hbm.at[pl.ds(off, TCH), :], ssem.at[bp])

    for k0 in range(2):
        make_gather(k0, k0 % NBG).start()
        make_poscopy(k0, k0 % NBP).start()

    def outer(g, _):
        for s in range(STEP):
            k = g * STEP + s
            bg = s % NBG
            bp = s % NBP
            make_gather(k, bg).wait()
            make_poscopy(k, bp).wait()

            @pl.when(k > 0)
            def _():
                make_store(k - 1, (s - 1) % NBP).wait()

            @pl.when(k + 2 < CPW)
            def _():
                make_poscopy(k + 2, (s + 2) % NBP).start()

            def add_row(r, _):
                for c in range(D // L):
                    sl = pl.ds(c * L, L)
                    pos_v[bp, r, sl] = pos_v[bp, r, sl] + rows_in[bg, r, sl]
                return 0

            lax.fori_loop(0, TCH, add_row, 0)

            @pl.when(k + 2 < CPW)
            def _():
                make_gather(k + 2, bg).start()

            make_store(k, bp).start()
        return 0

    lax.fori_loop(0, CPW // STEP, outer, 0)

    make_store(CPW - 1, (STEP - 1) % NBP).wait()


def kernel(idx, token_table, pos_table):
    idx2 = idx.reshape(NW, CPW, TCH)
    pos2 = jnp.concatenate([pos_table, pos_table], axis=0)
    out = _embed(idx2, token_table, pos2)
    return out.reshape(B, T, D)


# speedup vs baseline: 1.7036x; 1.7036x over previous
import functools

import jax
import jax.numpy as jnp
from jax import lax
from jax.experimental import pallas as pl
from jax.experimental.pallas import tpu as pltpu
from jax.experimental.pallas import tpu_sc as plsc

NC, NS, L = 2, 16, 16
NW = NC * NS
B, T, D = 256, 300, 512
SPW = B // NW
TCH = 32
NJT = T // TCH
TAIL = T - NJT * TCH
NB = 4

_mesh = plsc.VectorSubcoreMesh(core_axis_name="c", subcore_axis_name="s")


@functools.partial(
    pl.kernel,
    out_type=jax.ShapeDtypeStruct((B, T, D), jnp.float32),
    mesh=_mesh,
    scratch_types=[
        pltpu.VMEM((SPW * NJT, TCH), jnp.int32),
        pltpu.VMEM((SPW, 8), jnp.int32),
        pltpu.VMEM((SPW, 4), jnp.int32),
        pltpu.VMEM((NB, TCH, D), jnp.float32),
        pltpu.VMEM((TCH, D), jnp.float32),
        pltpu.VMEM((2, 8, D), jnp.float32),
        pltpu.VMEM((2, 4, D), jnp.float32),
        pltpu.VMEM((8, D), jnp.float32),
        pltpu.VMEM((4, D), jnp.float32),
        pltpu.SemaphoreType.DMA((NB,)),
        pltpu.SemaphoreType.DMA((NB,)),
        pltpu.SemaphoreType.DMA((2,)),
        pltpu.SemaphoreType.DMA((2,)),
        pltpu.SemaphoreType.DMA((2,)),
        pltpu.SemaphoreType.DMA((2,)),
    ],
)
def _embed(idx_hbm, idxta_hbm, idxtb_hbm, tok_hbm, pos_hbm, out_hbm,
           idx_v, idxta_v, idxtb_v, rows_v, pos_v, rows_a, rows_b,
           pos_a, pos_b, gsem, ssem, agsem, assem, bgsem, bssem):
    sid = lax.axis_index("s")
    wid = sid * NC + lax.axis_index("c")
    seq0 = pl.multiple_of(wid * SPW, 8)

    nrow = SPW * NJT
    pltpu.sync_copy(idx_hbm.at[pl.ds(pl.multiple_of(wid * nrow, 8), nrow), :],
                    idx_v)
    pltpu.sync_copy(idxta_hbm.at[pl.ds(seq0, SPW), :], idxta_v)
    pltpu.sync_copy(idxtb_hbm.at[pl.ds(seq0, SPW), :], idxtb_v)

    def make_gather(s, jt, b):
        return pltpu.make_async_copy(
            tok_hbm.at[idx_v.at[s * NJT + jt]], rows_v.at[b], gsem.at[b])

    def make_store(s, t0, b):
        return pltpu.make_async_copy(
            rows_v.at[b], out_hbm.at[seq0 + s, pl.ds(t0, TCH), :], ssem.at[b])

    def make_tgather(s, b, idxr, rows, sem):
        return pltpu.make_async_copy(
            tok_hbm.at[idxr.at[s]], rows.at[b], sem.at[b])

    def make_tstore(s, b, rows, t0, sz, sem):
        return pltpu.make_async_copy(
            rows.at[b], out_hbm.at[seq0 + s, pl.ds(t0, sz), :], sem.at[b])

    def add_rows(rv, pv, b, sz):
        def add_row(r, _):
            for c in range(D // L):
                sl = pl.ds(c * L, L)
                rv[b, r, sl] = rv[b, r, sl] + pv[r, sl]
            return 0
        lax.fori_loop(0, sz, add_row, 0)

    def jt_body(jt, _):
        t0 = pl.multiple_of(jt * TCH, 8)

        @pl.when(jt > 0)
        def _():
            for b in range(2):
                make_store(0, 0, b).wait()
        for s0 in range(2):
            make_gather(s0, jt, s0).start()

        pltpu.sync_copy(pos_hbm.at[pl.ds(t0, TCH), :], pos_v)

        for s in range(SPW):
            b = s % NB
            make_gather(s, jt, b).wait()
            add_rows(rows_v, pos_v, b, TCH)
            make_store(s, t0, b).start()
            if s + 2 < SPW:
                @pl.when((jt > 0) | (s >= 2))
                def _():
                    make_store(0, 0, (s + 2) % NB).wait()
                make_gather(s + 2, jt, (s + 2) % NB).start()
        return 0

    lax.fori_loop(0, NJT, jt_body, 0)

    for b in range(NB):
        make_store(0, 0, b).wait()

    TA, TB = NJT * TCH, NJT * TCH + 8
    pltpu.sync_copy(pos_hbm.at[pl.ds(TA, 8), :], pos_a)
    pltpu.sync_copy(pos_hbm.at[pl.ds(TB, 4), :], pos_b)
    for s0 in range(2):
        make_tgather(s0, s0, idxta_v, rows_a, agsem).start()
        make_tgather(s0, s0, idxtb_v, rows_b, bgsem).start()
    for s in range(SPW):
        b = s % 2
        make_tgather(s, b, idxta_v, rows_a, agsem).wait()
        make_tgather(s, b, idxtb_v, rows_b, bgsem).wait()
        add_rows(rows_a, pos_a, b, 8)
        add_rows(rows_b, pos_b, b, 4)
        make_tstore(s, b, rows_a, TA, 8, assem).start()
        make_tstore(s, b, rows_b, TB, 4, bssem).start()
        if s + 2 < SPW:
            make_tstore(0, b, rows_a, TA, 8, assem).wait()
            make_tstore(0, b, rows_b, TB, 4, bssem).wait()
            make_tgather(s + 2, b, idxta_v, rows_a, agsem).start()
            make_tgather(s + 2, b, idxtb_v, rows_b, bgsem).start()
    for b in range(2):
        make_tstore(0, b, rows_a, TA, 8, assem).wait()
        make_tstore(0, b, rows_b, TB, 4, bssem).wait()


def kernel(idx, token_table, pos_table):
    idx_main = idx[:, :NJT * TCH].reshape(B * NJT, TCH)
    idx_ta = idx[:, NJT * TCH:NJT * TCH + 8]
    idx_tb = idx[:, NJT * TCH + 8:]
    return _embed(idx_main, idx_ta, idx_tb, token_table, pos_table)


# speedup vs baseline: 2.1817x; 1.2806x over previous
import functools

import jax
import jax.numpy as jnp
from jax import lax
from jax.experimental import pallas as pl
from jax.experimental.pallas import tpu as pltpu
from jax.experimental.pallas import tpu_sc as plsc

NC, NS, L = 2, 16, 16
NW = NC * NS
B, T, D = 256, 300, 512
BB = 64
NBB = B // BB
UNITS = T * NBB
NB = 3
JMAX = 39

_mesh = plsc.VectorSubcoreMesh(core_axis_name="c", subcore_axis_name="s")


@functools.partial(
    pl.kernel,
    out_type=jax.ShapeDtypeStruct((T, B, D), jnp.float32),
    mesh=_mesh,
    scratch_types=[
        pltpu.VMEM((NB, BB, D), jnp.float32),
        pltpu.VMEM((NB, 1, D), jnp.float32),
        pltpu.VMEM((NB, 1, BB), jnp.int32),
        pltpu.SemaphoreType.DMA((NB,)),
        pltpu.SemaphoreType.DMA((NB,)),
        pltpu.SemaphoreType.DMA((NB,)),
        pltpu.SemaphoreType.DMA((NB,)),
    ],
)
def _embed(idxf_hbm, tok_hbm, posf_hbm, out_hbm,
           rows_v, pos_v, idx_v, gsem, psem, isem, ssem):
    wid = lax.axis_index("s") * NC + lax.axis_index("c")

    def unit(j):
        u = wid + j * NW
        return u // NBB, BB * lax.rem(u, NBB)

    def valid(j):
        return wid + j * NW < UNITS

    def make_idxcopy(j, b):
        t, b0 = unit(j)
        off = pl.multiple_of(t * B + b0, 8)
        return pltpu.make_async_copy(
            idxf_hbm.at[pl.ds(off, BB)], idx_v.at[b, 0], isem.at[b])

    def make_poscopy(j, b):
        t, _ = unit(j)
        off = pl.multiple_of(t * D, 8)
        return pltpu.make_async_copy(
            posf_hbm.at[pl.ds(off, D)], pos_v.at[b, 0], psem.at[b])

    def make_gather(b):
        return pltpu.make_async_copy(
            tok_hbm.at[idx_v.at[b, 0]], rows_v.at[b], gsem.at[b])

    def make_store(j, b):
        t, b0 = unit(j)
        return pltpu.make_async_copy(
            rows_v.at[b],
            out_hbm.at[t, pl.ds(pl.multiple_of(b0, 8), BB), :], ssem.at[b])

    for j0 in range(2):
        make_idxcopy(j0, j0).start()
        make_poscopy(j0, j0).start()
    make_idxcopy(0, 0).wait()
    make_gather(0).start()

    def outer(g, _):
        for s in range(NB):
            j = g * NB + s
            b = s

            @pl.when(valid(j))
            def _():
                make_gather(b).wait()
                make_poscopy(j, b).wait()

                for c in range(D // L):
                    sl = pl.ds(c * L, L)
                    pc = pos_v[b, 0, sl]

                    def add4(r4, _, sl=sl, pc=pc, b=b):
                        r = r4 * 4
                        for i in range(4):
                            rows_v[b, r + i, sl] = rows_v[b, r + i, sl] + pc
                        return 0

                    lax.fori_loop(0, BB // 4, add4, 0)

            @pl.when((j >= 1) & valid(j - 1))
            def _():
                make_store(j - 1, (s - 1) % NB).wait()

            @pl.when(valid(j))
            def _():
                make_store(j, b).start()

            @pl.when(valid(j + 2))
            def _():
                make_idxcopy(j + 2, (s + 2) % NB).start()
                make_poscopy(j + 2, (s + 2) % NB).start()

            @pl.when(valid(j + 1))
            def _():
                make_idxcopy(j + 1, (s + 1) % NB).wait()
                make_gather((s + 1) % NB).start()
        return 0

    lax.fori_loop(0, JMAX // NB, outer, 0)


def kernel(idx, token_table, pos_table):
    idxf = idx.T.reshape(T * B)
    posf = pos_table.reshape(T * D)
    out = _embed(idxf, token_table, posf)
    return out.transpose(1, 0, 2)


# speedup vs baseline: 3.2690x; 1.4984x over previous
import functools

import jax
import jax.numpy as jnp
from jax import lax
from jax.experimental import pallas as pl
from jax.experimental.pallas import tpu as pltpu
from jax.experimental.pallas import tpu_sc as plsc

NC, NS, L = 2, 16, 16
NW = NC * NS
B, T, D = 256, 300, 512
BB = 64
NBB = B // BB
UNITS = T * NBB
NB = 3
JMAX = 39

_mesh = plsc.VectorSubcoreMesh(core_axis_name="c", subcore_axis_name="s")


@functools.partial(
    pl.kernel,
    out_type=jax.ShapeDtypeStruct((T, B, D), jnp.float32),
    mesh=_mesh,
    scratch_types=[
        pltpu.VMEM((NB, BB, D), jnp.float32),
        pltpu.VMEM((NB, 1, D), jnp.float32),
        pltpu.VMEM((NB, 1, BB), jnp.int32),
        pltpu.SemaphoreType.DMA((NB,)),
        pltpu.SemaphoreType.DMA((NB,)),
        pltpu.SemaphoreType.DMA((NB,)),
        pltpu.SemaphoreType.DMA((NB,)),
    ],
)
def _embed(idxf_hbm, tok_hbm, posf_hbm, out_hbm,
           rows_v, pos_v, idx_v, gsem, psem, isem, ssem):
    wid = lax.axis_index("s") * NC + lax.axis_index("c")

    def unit(j):
        u = wid + j * NW
        return u // NBB, BB * lax.rem(u, NBB)

    def valid(j):
        return wid + j * NW < UNITS

    def make_idxcopy(j, b):
        t, b0 = unit(j)
        off = pl.multiple_of(t * B + b0, 8)
        return pltpu.make_async_copy(
            idxf_hbm.at[pl.ds(off, BB)], idx_v.at[b, 0], isem.at[b])

    def make_poscopy(j, b):
        t, _ = unit(j)
        off = pl.multiple_of(t * D, 8)
        return pltpu.make_async_copy(
            posf_hbm.at[pl.ds(off, D)], pos_v.at[b, 0], psem.at[b])

    def make_gather(b):
        return pltpu.make_async_copy(
            tok_hbm.at[idx_v.at[b, 0]], rows_v.at[b], gsem.at[b])

    def make_store(j, b):
        t, b0 = unit(j)
        return pltpu.make_async_copy(
            rows_v.at[b],
            out_hbm.at[t, pl.ds(pl.multiple_of(b0, 8), BB), :], ssem.at[b])

    for j0 in range(NB):
        make_idxcopy(j0, j0).start()
        make_poscopy(j0, j0).start()
    for j0 in range(2):
        make_idxcopy(j0, j0).wait()
        make_gather(j0).start()

    def outer(g, _):
        for s in range(NB):
            j = g * NB + s
            b = s

            @pl.when(valid(j))
            def _():
                make_gather(b).wait()
                make_poscopy(j, b).wait()

                for c in range(D // L):
                    sl = pl.ds(c * L, L)
                    pc = pos_v[b, 0, sl]

                    def add4(r4, _, sl=sl, pc=pc, b=b):
                        r = r4 * 4
                        for i in range(4):
                            rows_v[b, r + i, sl] = rows_v[b, r + i, sl] + pc
                        return 0

                    lax.fori_loop(0, BB // 4, add4, 0)

            @pl.when((j >= 1) & valid(j - 1))
            def _():
                make_store(j - 1, (s - 1) % NB).wait()

            @pl.when(valid(j))
            def _():
                make_store(j, b).start()

            @pl.when(valid(j + 3))
            def _():
                make_idxcopy(j + 3, b).start()
                make_poscopy(j + 3, b).start()

            @pl.when(valid(j + 2))
            def _():
                make_idxcopy(j + 2, (s + 2) % NB).wait()
                make_gather((s + 2) % NB).start()
        return 0

    lax.fori_loop(0, JMAX // NB, outer, 0)


def kernel(idx, token_table, pos_table):
    idxf = idx.T.reshape(T * B)
    posf = pos_table.reshape(T * D)
    out = _embed(idxf, token_table, posf)
    return out.transpose(1, 0, 2)
